# baseline (speedup 1.0000x reference)
import jax
import jax.numpy as jnp
from jax import lax
from jax.experimental import pallas as pl
from jax.experimental.pallas import tpu as pltpu
from jax.experimental.pallas import tpu_sc as plsc

NC, NS, L = 2, 16, 16
NW = NC * NS
B, S, H = 4, 8192, 1024
PPW = S // NW
C = 8
NJ = PPW // C
K = H // L


def _sc_body(ids_hbm, wte_hbm, wpe_hbm, out_hbm, idxs,
             g0, g1, g2, wb0, wb1, wb2, sg0, sg1, sg2, sw0, sw1, sw2,
             so00, so01, so02, so03, so10, so11, so12, so13,
             so20, so21, so22, so23, sidx):
    w = lax.axis_index("s") * NC + lax.axis_index("c")
    pos0 = w * PPW
    gbuf, wb = (g0, g1, g2), (wb0, wb1, wb2)
    sg, sw = (sg0, sg1, sg2), (sw0, sw1, sw2)
    so = ((so00, so01, so02, so03), (so10, so11, so12, so13),
          (so20, so21, so22, so23))

    pltpu.async_copy(ids_hbm.at[w], idxs, sidx).wait()

    def fire_inputs(j, p):
        pltpu.async_copy(wpe_hbm.at[pl.ds(pos0 + j * C, C)], wb[p], sw[p])
        pltpu.async_copy(wte_hbm.at[idxs.at[j]], gbuf[p], sg[p])

    def wait_inputs(p):
        pltpu.make_async_copy(wpe_hbm.at[pl.ds(0, C)], wb[p], sw[p]).wait()
        pltpu.make_async_copy(wte_hbm.at[idxs.at[0]], gbuf[p],
                              sg[p]).wait()

    def wait_outs(p):
        for b in range(B):
            pltpu.make_async_copy(gbuf[p].at[pl.ds(0, C)],
                                  out_hbm.at[0, pl.ds(0, C)],
                                  so[p][b]).wait()

    def do_step(j, p, first=False, last=False):
        if not first:
            wait_outs((p + 1) % 3)
        if not last:
            fire_inputs(j + 1, (p + 1) % 3)
        wait_inputs(p)

        @plsc.parallel_loop(0, C)
        def _(r):
            for k in range(K):
                sl = pl.ds(k * L, L)
                v = wb[p][r, sl]
                for b in range(B):
                    plsc.addupdate(gbuf[p].at[b * C + r, sl], v)

        for b in range(B):
            pltpu.async_copy(gbuf[p].at[pl.ds(b * C, C)],
                             out_hbm.at[b, pl.ds(pos0 + j * C, C)],
                             so[p][b])

    fire_inputs(0, 0)
    do_step(0, 0, first=True)
    do_step(1, 1, first=True)

    def jj_body(t, _):
        for u in range(3):
            do_step(2 + 3 * t + u, (2 + u) % 3)
        return 0
    lax.fori_loop(0, (NJ - 5) // 3, jj_body, 0)

    do_step(NJ - 3, (NJ - 3) % 3)
    do_step(NJ - 2, (NJ - 2) % 3)
    do_step(NJ - 1, (NJ - 1) % 3, last=True)
    wait_outs((NJ - 2) % 3)
    wait_outs((NJ - 1) % 3)


def _sc_call(ids_r, wte, wpe):
    mesh = plsc.VectorSubcoreMesh(core_axis_name="c", subcore_axis_name="s",
                                  num_cores=NC, num_subcores=NS)
    sem = pltpu.SemaphoreType.DMA
    f = pl.kernel(
        _sc_body,
        out_type=jax.ShapeDtypeStruct((B, S, H), jnp.float32),
        mesh=mesh,
        scratch_types=(
            [pltpu.VMEM((NJ, B * C), jnp.int32)]
            + [pltpu.VMEM((B * C, H), jnp.float32) for _ in range(3)]
            + [pltpu.VMEM((C, H), jnp.float32) for _ in range(3)]
            + [sem] * 19
        ),
    )
    return f(ids_r, wte, wpe)


@jax.jit
def kernel(input_ids, wte, wpe):
    ids = input_ids.astype(jnp.int32)
    ids_r = (ids.reshape(B, NW, NJ, C)
                .transpose(1, 2, 0, 3)
                .reshape(NW, NJ, B * C))
    return _sc_call(ids_r, wte, wpe)


# speedup vs baseline: 1.0339x; 1.0339x over previous
import jax
import jax.numpy as jnp
from jax import lax
from jax.experimental import pallas as pl
from jax.experimental.pallas import tpu as pltpu
from jax.experimental.pallas import tpu_sc as plsc

NC, NS, L = 2, 16, 16
NW = NC * NS
B, S, H = 4, 8192, 1024
PPW = S // NW
C = 8
NJ = PPW // C
K = H // L


def _sc_body(ids_hbm, wte_hbm, wpe_hbm, out_hbm, idxs,
             g0, g1, wb0, wb1, sg0, sg1, sw0, sw1,
             so00, so01, so02, so03, so10, so11, so12, so13, sidx):
    w = lax.axis_index("s") * NC + lax.axis_index("c")
    pos0 = w * PPW
    gbuf, wb = (g0, g1), (wb0, wb1)
    sg, sw = (sg0, sg1), (sw0, sw1)
    so = ((so00, so01, so02, so03), (so10, so11, so12, so13))

    pltpu.async_copy(ids_hbm.at[w], idxs, sidx).wait()

    def fire_inputs(j, p):
        pltpu.async_copy(wpe_hbm.at[pl.ds(pos0 + j * C, C)], wb[p], sw[p])
        pltpu.async_copy(wte_hbm.at[idxs.at[j]], gbuf[p], sg[p])

    def wait_inputs(p):
        pltpu.make_async_copy(wpe_hbm.at[pl.ds(0, C)], wb[p], sw[p]).wait()
        pltpu.make_async_copy(wte_hbm.at[idxs.at[0]], gbuf[p],
                              sg[p]).wait()

    def wait_outs(p):
        for b in range(B):
            pltpu.make_async_copy(gbuf[p].at[pl.ds(0, C)],
                                  out_hbm.at[0, pl.ds(0, C)],
                                  so[p][b]).wait()

    def do_step(j, p, first=False, last=False):
        if not last:
            if not first:
                wait_outs(1 - p)
            fire_inputs(j + 1, 1 - p)
        wait_inputs(p)

        @plsc.parallel_loop(0, C, unroll=2)
        def _(r):
            for k in range(K):
                sl = pl.ds(k * L, L)
                v = wb[p][r, sl]
                for b in range(B):
                    plsc.addupdate(gbuf[p].at[b * C + r, sl], v)

        for b in range(B):
            pltpu.async_copy(gbuf[p].at[pl.ds(b * C, C)],
                             out_hbm.at[b, pl.ds(pos0 + j * C, C)],
                             so[p][b])

    fire_inputs(0, 0)
    do_step(0, 0, first=True)

    def jj_body(jj, _):
        for j2 in range(2):
            do_step(1 + 2 * jj + j2, (1 + j2) % 2)
        return 0
    lax.fori_loop(0, (NJ - 2) // 2, jj_body, 0)

    do_step(NJ - 1, (NJ - 1) % 2, last=True)
    wait_outs(0)
    wait_outs(1)


def _sc_call(ids_r, wte, wpe):
    mesh = plsc.VectorSubcoreMesh(core_axis_name="c", subcore_axis_name="s",
                                  num_cores=NC, num_subcores=NS)
    sem = pltpu.SemaphoreType.DMA
    f = pl.kernel(
        _sc_body,
        out_type=jax.ShapeDtypeStruct((B, S, H), jnp.float32),
        mesh=mesh,
        scratch_types=(
            [pltpu.VMEM((NJ, B * C), jnp.int32),
             pltpu.VMEM((B * C, H), jnp.float32),
             pltpu.VMEM((B * C, H), jnp.float32),
             pltpu.VMEM((C, H), jnp.float32),
             pltpu.VMEM((C, H), jnp.float32)]
            + [sem] * 13
        ),
    )
    return f(ids_r, wte, wpe)


@jax.jit
def kernel(input_ids, wte, wpe):
    ids = input_ids.astype(jnp.int32)
    ids_r = (ids.reshape(B, NW, NJ, C)
                .transpose(1, 2, 0, 3)
                .reshape(NW, NJ, B * C))
    return _sc_call(ids_r, wte, wpe)
